# baseline (speedup 1.0000x reference)
import jax
import jax.numpy as jnp
from jax.experimental import pallas as pl
from jax.experimental.pallas import tpu as pltpu

_R = 64
_NG = _R // 2 + 1


def _gelu_exact(v):
    return 0.5 * v * (1.0 + jax.lax.erf(v * 0.7071067811865476))


def _body(x_hbm, w1a_ref, b1a_ref, w2a_ref, b2a_ref,
          w1b_ref, b1b_ref, w2b_ref, b2b_ref,
          ws1_hbm, bs1_ref, ws3_hbm, bs3_ref, ws2_hbm, bs2_ref,
          out_hbm,
          xbuf, obufA, obufB, k0buf, ws1_ref, ws3_ref, ws2_ref,
          in_sems, oa_sems, ob_sems, ws_sems):
    f32 = jnp.float32
    g = pl.program_id(0)
    ca_prev = jax.lax.rem(10 * g + 27, _R)
    ca = jax.lax.rem(10 * g, _R)
    cb = jax.lax.rem(10 * g + 37, _R)
    sa_prev = jax.lax.rem(2 * g + 4, 5)
    sa = jax.lax.rem(2 * g, 5)
    sb = jax.lax.rem(2 * g + 1, 5)
    q = jax.lax.rem(g, 2)

    def in_copy(c, slot):
        return pltpu.make_async_copy(
            x_hbm.at[:, c, :], xbuf.at[slot], in_sems.at[slot])

    def ws_copies():
        return (pltpu.make_async_copy(ws1_hbm, ws1_ref, ws_sems.at[0]),
                pltpu.make_async_copy(ws3_hbm, ws3_ref, ws_sems.at[1]),
                pltpu.make_async_copy(ws2_hbm, ws2_ref, ws_sems.at[2]))

    @pl.when(g == 0)
    def _():
        in_copy(ca, sa).start()
        in_copy(cb, sb).start()
        for cp in ws_copies():
            cp.start()
    in_copy(ca, sa).wait()
    in_copy(cb, sb).wait()

    @pl.when(g < _NG - 1)
    def _():
        in_copy(jax.lax.rem(10 * g + 10, _R), jax.lax.rem(2 * g + 2, 5)).start()
        in_copy(jax.lax.rem(10 * g + 47, _R), jax.lax.rem(2 * g + 3, 5)).start()

    xa = xbuf[sa_prev]
    xb = xbuf[sa]
    xc = xbuf[sb]
    j = xb.shape[0]

    xeA = jnp.concatenate([xa, xb], axis=0)
    hA = _gelu_exact(jnp.dot(xeA, w1a_ref[0], preferred_element_type=f32)
                     + b1a_ref[0])
    eA = jnp.dot(hA, w2a_ref[0], preferred_element_type=f32) + b2a_ref[0]
    xeB = jnp.concatenate([xb, xc], axis=0)
    hB = _gelu_exact(jnp.dot(xeB, w1b_ref[0], preferred_element_type=f32)
                     + b1b_ref[0])
    eB = jnp.dot(hB, w2b_ref[0], preferred_element_type=f32) + b2b_ref[0]

    @pl.when(g == 1)
    def _():
        for cp in ws_copies():
            cp.wait()

    h1 = jnp.dot(xeA, ws1_ref[...], preferred_element_type=f32) + bs1_ref[...]
    h3 = jnp.dot(xeA, ws3_ref[...], preferred_element_type=f32) + bs3_ref[...]
    sh = jnp.dot(jax.nn.silu(h1) * h3, ws2_ref[...],
                 preferred_element_type=f32) + bs2_ref[...]

    k0_prev = k0buf[...]
    k0buf[...] = eB[j:, :]

    @pl.when(g > 0)
    def _():
        @pl.when(g >= 3)
        def _():
            pltpu.make_async_copy(obufA.at[q], out_hbm.at[:, ca_prev, :],
                                  oa_sems.at[q]).wait()
        obufA[q] = 0.5 * sh[:j, :] + 0.25 * (k0_prev + eA[:j, :])
        pltpu.make_async_copy(obufA.at[q], out_hbm.at[:, ca_prev, :],
                              oa_sems.at[q]).start()

    @pl.when(g >= 2)
    def _():
        pltpu.make_async_copy(obufB.at[q], out_hbm.at[:, ca, :],
                              ob_sems.at[q]).wait()
    obufB[q] = 0.5 * sh[j:, :] + 0.25 * (eA[j:, :] + eB[:j, :])
    pltpu.make_async_copy(obufB.at[q], out_hbm.at[:, ca, :],
                          ob_sems.at[q]).start()

    @pl.when(g == _NG - 1)
    def _():
        ca_prev1 = jax.lax.rem(10 * g + 17, _R)
        ca1 = jax.lax.rem(10 * g + 54, _R)
        pltpu.make_async_copy(obufA.at[q], out_hbm.at[:, ca_prev, :],
                              oa_sems.at[q]).wait()
        pltpu.make_async_copy(obufA.at[1 - q], out_hbm.at[:, ca_prev1, :],
                              oa_sems.at[1 - q]).wait()
        pltpu.make_async_copy(obufB.at[q], out_hbm.at[:, ca, :],
                              ob_sems.at[q]).wait()
        pltpu.make_async_copy(obufB.at[1 - q], out_hbm.at[:, ca1, :],
                              ob_sems.at[1 - q]).wait()


def kernel(x, t_emb, Ws1, bs1, Ws3, bs3, Ws2, bs2, W1, b1, W2, b2):
    B, T, C = x.shape
    N = B * T
    J = N // _R
    E, _, HR = W1.shape
    HS = Ws1.shape[1]
    f32 = jnp.float32

    x3 = x.reshape(J, _R, C)
    b1r = b1[:, None, :]
    b2r = b2[:, None, :]
    bs1r = bs1[None, :]
    bs3r = bs3[None, :]
    bs2r = bs2[None, :]

    out = pl.pallas_call(
        _body,
        grid=(_NG,),
        in_specs=[
            pl.BlockSpec(memory_space=pl.ANY),
            pl.BlockSpec((1, C, HR), lambda g: ((30 * g) % _R, 0, 0)),
            pl.BlockSpec((1, 1, HR), lambda g: ((30 * g) % _R, 0, 0)),
            pl.BlockSpec((1, HR, C), lambda g: ((30 * g) % _R, 0, 0)),
            pl.BlockSpec((1, 1, C), lambda g: ((30 * g) % _R, 0, 0)),
            pl.BlockSpec((1, C, HR), lambda g: ((30 * g + 47) % _R, 0, 0)),
            pl.BlockSpec((1, 1, HR), lambda g: ((30 * g + 47) % _R, 0, 0)),
            pl.BlockSpec((1, HR, C), lambda g: ((30 * g + 47) % _R, 0, 0)),
            pl.BlockSpec((1, 1, C), lambda g: ((30 * g + 47) % _R, 0, 0)),
            pl.BlockSpec(memory_space=pl.ANY),
            pl.BlockSpec((1, HS), lambda g: (0, 0)),
            pl.BlockSpec(memory_space=pl.ANY),
            pl.BlockSpec((1, HS), lambda g: (0, 0)),
            pl.BlockSpec(memory_space=pl.ANY),
            pl.BlockSpec((1, C), lambda g: (0, 0)),
        ],
        out_specs=pl.BlockSpec(memory_space=pl.ANY),
        out_shape=jax.ShapeDtypeStruct((J, _R, C), f32),
        scratch_shapes=[
            pltpu.VMEM((5, J, C), f32),
            pltpu.VMEM((2, J, C), f32),
            pltpu.VMEM((2, J, C), f32),
            pltpu.VMEM((J, C), f32),
            pltpu.VMEM((C, HS), f32),
            pltpu.VMEM((C, HS), f32),
            pltpu.VMEM((HS, C), f32),
            pltpu.SemaphoreType.DMA((5,)),
            pltpu.SemaphoreType.DMA((2,)),
            pltpu.SemaphoreType.DMA((2,)),
            pltpu.SemaphoreType.DMA((3,)),
        ],
    )(x3, W1, b1r, W2, b2r, W1, b1r, W2, b2r,
      Ws1, bs1r, Ws3, bs3r, Ws2, bs2r)

    return out.reshape(B, T, C)


# speedup vs baseline: 1.0280x; 1.0280x over previous
import jax
import jax.numpy as jnp
from jax.experimental import pallas as pl
from jax.experimental.pallas import tpu as pltpu

_R = 64
_NG = _R // 2 + 1


def _gelu_exact(v):
    return 0.5 * v * (1.0 + jax.lax.erf(v * 0.7071067811865476))


def _body(x_hbm, w1a_ref, b1a_ref, w2a_ref, b2a_ref,
          w1b_ref, b1b_ref, w2b_ref, b2b_ref,
          ws1_ref, bs1_ref, ws3_ref, bs3_ref, ws2_ref, bs2_ref,
          out_hbm,
          xbuf, obufA, obufB, k0buf, in_sems, oa_sems, ob_sems):
    f32 = jnp.float32
    g = pl.program_id(0)
    ca_prev = jax.lax.rem(10 * g + 27, _R)
    ca = jax.lax.rem(10 * g, _R)
    cb = jax.lax.rem(10 * g + 37, _R)
    sa_prev = jax.lax.rem(2 * g + 4, 5)
    sa = jax.lax.rem(2 * g, 5)
    sb = jax.lax.rem(2 * g + 1, 5)
    q = jax.lax.rem(g, 2)

    def in_copy(c, slot):
        return pltpu.make_async_copy(
            x_hbm.at[:, c, :], xbuf.at[slot], in_sems.at[slot])

    @pl.when(g == 0)
    def _():
        in_copy(ca, sa).start()
        in_copy(cb, sb).start()
    in_copy(ca, sa).wait()
    in_copy(cb, sb).wait()

    @pl.when(g < _NG - 1)
    def _():
        in_copy(jax.lax.rem(10 * g + 10, _R), jax.lax.rem(2 * g + 2, 5)).start()
        in_copy(jax.lax.rem(10 * g + 47, _R), jax.lax.rem(2 * g + 3, 5)).start()

    xa = xbuf[sa_prev]
    xb = xbuf[sa]
    xc = xbuf[sb]
    j = xb.shape[0]

    xeA = jnp.concatenate([xa, xb], axis=0)
    hA = _gelu_exact(jnp.dot(xeA, w1a_ref[0], preferred_element_type=f32)
                     + b1a_ref[0])
    eA = jnp.dot(hA, w2a_ref[0], preferred_element_type=f32) + b2a_ref[0]
    xeB = jnp.concatenate([xb, xc], axis=0)
    hB = _gelu_exact(jnp.dot(xeB, w1b_ref[0], preferred_element_type=f32)
                     + b1b_ref[0])
    eB = jnp.dot(hB, w2b_ref[0], preferred_element_type=f32) + b2b_ref[0]

    h1 = jnp.dot(xeA, ws1_ref[...], preferred_element_type=f32) + bs1_ref[...]
    h3 = jnp.dot(xeA, ws3_ref[...], preferred_element_type=f32) + bs3_ref[...]
    sh = jnp.dot(jax.nn.silu(h1) * h3, ws2_ref[...],
                 preferred_element_type=f32) + bs2_ref[...]

    k0_prev = k0buf[...]
    k0buf[...] = eB[j:, :]

    @pl.when(g > 0)
    def _():
        @pl.when(g >= 3)
        def _():
            pltpu.make_async_copy(obufA.at[q], out_hbm.at[:, ca_prev, :],
                                  oa_sems.at[q]).wait()
        obufA[q] = 0.5 * sh[:j, :] + 0.25 * (k0_prev + eA[:j, :])
        pltpu.make_async_copy(obufA.at[q], out_hbm.at[:, ca_prev, :],
                              oa_sems.at[q]).start()

    @pl.when(g >= 2)
    def _():
        pltpu.make_async_copy(obufB.at[q], out_hbm.at[:, ca, :],
                              ob_sems.at[q]).wait()
    obufB[q] = 0.5 * sh[j:, :] + 0.25 * (eA[j:, :] + eB[:j, :])
    pltpu.make_async_copy(obufB.at[q], out_hbm.at[:, ca, :],
                          ob_sems.at[q]).start()

    @pl.when(g == _NG - 1)
    def _():
        ca_prev1 = jax.lax.rem(10 * g + 17, _R)
        ca1 = jax.lax.rem(10 * g + 54, _R)
        pltpu.make_async_copy(obufA.at[q], out_hbm.at[:, ca_prev, :],
                              oa_sems.at[q]).wait()
        pltpu.make_async_copy(obufA.at[1 - q], out_hbm.at[:, ca_prev1, :],
                              oa_sems.at[1 - q]).wait()
        pltpu.make_async_copy(obufB.at[q], out_hbm.at[:, ca, :],
                              ob_sems.at[q]).wait()
        pltpu.make_async_copy(obufB.at[1 - q], out_hbm.at[:, ca1, :],
                              ob_sems.at[1 - q]).wait()


def kernel(x, t_emb, Ws1, bs1, Ws3, bs3, Ws2, bs2, W1, b1, W2, b2):
    B, T, C = x.shape
    N = B * T
    J = N // _R
    E, _, HR = W1.shape
    HS = Ws1.shape[1]
    f32 = jnp.float32

    x3 = x.reshape(J, _R, C)
    b1r = b1[:, None, :]
    b2r = b2[:, None, :]
    bs1r = bs1[None, :]
    bs3r = bs3[None, :]
    bs2r = bs2[None, :]

    out = pl.pallas_call(
        _body,
        grid=(_NG,),
        in_specs=[
            pl.BlockSpec(memory_space=pl.ANY),
            pl.BlockSpec((1, C, HR), lambda g: ((30 * g) % _R, 0, 0)),
            pl.BlockSpec((1, 1, HR), lambda g: ((30 * g) % _R, 0, 0)),
            pl.BlockSpec((1, HR, C), lambda g: ((30 * g) % _R, 0, 0)),
            pl.BlockSpec((1, 1, C), lambda g: ((30 * g) % _R, 0, 0)),
            pl.BlockSpec((1, C, HR), lambda g: ((30 * g + 47) % _R, 0, 0)),
            pl.BlockSpec((1, 1, HR), lambda g: ((30 * g + 47) % _R, 0, 0)),
            pl.BlockSpec((1, HR, C), lambda g: ((30 * g + 47) % _R, 0, 0)),
            pl.BlockSpec((1, 1, C), lambda g: ((30 * g + 47) % _R, 0, 0)),
            pl.BlockSpec((C, HS), lambda g: (0, 0)),
            pl.BlockSpec((1, HS), lambda g: (0, 0)),
            pl.BlockSpec((C, HS), lambda g: (0, 0)),
            pl.BlockSpec((1, HS), lambda g: (0, 0)),
            pl.BlockSpec((HS, C), lambda g: (0, 0)),
            pl.BlockSpec((1, C), lambda g: (0, 0)),
        ],
        out_specs=pl.BlockSpec(memory_space=pl.ANY),
        out_shape=jax.ShapeDtypeStruct((J, _R, C), f32),
        scratch_shapes=[
            pltpu.VMEM((5, J, C), f32),
            pltpu.VMEM((2, J, C), f32),
            pltpu.VMEM((2, J, C), f32),
            pltpu.VMEM((J, C), f32),
            pltpu.SemaphoreType.DMA((5,)),
            pltpu.SemaphoreType.DMA((2,)),
            pltpu.SemaphoreType.DMA((2,)),
        ],
    )(x3, W1, b1r, W2, b2r, W1, b1r, W2, b2r,
      Ws1, bs1r, Ws3, bs3r, Ws2, bs2r)

    return out.reshape(B, T, C)


# speedup vs baseline: 1.0322x; 1.0041x over previous
import jax
import jax.numpy as jnp
from jax.experimental import pallas as pl
from jax.experimental.pallas import tpu as pltpu

_R = 64
_NG = _R // 2 + 1


def _gelu_exact(v):
    return 0.5 * v * (1.0 + jax.lax.erf(v * 0.7071067811865476))


def _body(x_hbm, w1a_ref, b1a_ref, w2a_ref, b2a_ref,
          w1b_ref, b1b_ref, w2b_ref, b2b_ref,
          ws1_ref, bs1_ref, ws3_ref, bs3_ref, ws2_ref, bs2_ref,
          out_hbm,
          xbuf, obufA, obufB, k0buf, in_sems, oa_sems, ob_sems):
    f32 = jnp.float32
    g = pl.program_id(0)
    ca_prev = jax.lax.rem(10 * g + 27, _R)
    ca = jax.lax.rem(10 * g, _R)
    cb = jax.lax.rem(10 * g + 37, _R)
    sa_prev = jax.lax.rem(2 * g + 4, 5)
    sa = jax.lax.rem(2 * g, 5)
    sb = jax.lax.rem(2 * g + 1, 5)
    q = jax.lax.rem(g, 2)

    def in_copy(c, slot):
        return pltpu.make_async_copy(
            x_hbm.at[:, c, :], xbuf.at[slot], in_sems.at[slot])

    @pl.when(g == 0)
    def _():
        in_copy(ca, sa).start()
        in_copy(cb, sb).start()
    in_copy(ca, sa).wait()
    in_copy(cb, sb).wait()

    @pl.when(g < _NG - 1)
    def _():
        in_copy(jax.lax.rem(10 * g + 10, _R), jax.lax.rem(2 * g + 2, 5)).start()
        in_copy(jax.lax.rem(10 * g + 47, _R), jax.lax.rem(2 * g + 3, 5)).start()

    xa = xbuf[sa_prev]
    xb = xbuf[sa]
    xc = xbuf[sb]
    j = xb.shape[0]

    xeA = jnp.concatenate([xa, xb], axis=0)
    hA = _gelu_exact(jnp.dot(xeA, w1a_ref[0], preferred_element_type=f32)
                     + b1a_ref[0])
    eA = jnp.dot(hA, w2a_ref[0], preferred_element_type=f32) + b2a_ref[0]
    xeB = jnp.concatenate([xb, xc], axis=0)
    hB = _gelu_exact(jnp.dot(xeB, w1b_ref[0], preferred_element_type=f32)
                     + b1b_ref[0])
    eB = jnp.dot(hB, w2b_ref[0], preferred_element_type=f32) + b2b_ref[0]

    h1 = jnp.dot(xeA, ws1_ref[...], preferred_element_type=f32) + bs1_ref[...]
    h3 = jnp.dot(xeA, ws3_ref[...], preferred_element_type=f32) + bs3_ref[...]
    sh = jnp.dot(jax.nn.silu(h1) * h3, ws2_ref[...],
                 preferred_element_type=f32) + bs2_ref[...]

    k0_prev = k0buf[...]
    k0buf[...] = eB[j:, :]

    @pl.when(g >= 2)
    def _():
        pltpu.make_async_copy(obufA.at[q], out_hbm.at[:, ca_prev, :],
                              oa_sems.at[q]).wait()
    obufA[q] = 0.5 * sh[:j, :] + 0.25 * (k0_prev + eA[:j, :])
    pltpu.make_async_copy(obufA.at[q], out_hbm.at[:, ca_prev, :],
                          oa_sems.at[q]).start()

    @pl.when(g >= 2)
    def _():
        pltpu.make_async_copy(obufB.at[q], out_hbm.at[:, ca, :],
                              ob_sems.at[q]).wait()
    obufB[q] = 0.5 * sh[j:, :] + 0.25 * (eA[j:, :] + eB[:j, :])
    pltpu.make_async_copy(obufB.at[q], out_hbm.at[:, ca, :],
                          ob_sems.at[q]).start()

    @pl.when(g == _NG - 1)
    def _():
        ca_prev1 = jax.lax.rem(10 * g + 17, _R)
        ca1 = jax.lax.rem(10 * g + 54, _R)
        pltpu.make_async_copy(obufA.at[q], out_hbm.at[:, ca_prev, :],
                              oa_sems.at[q]).wait()
        pltpu.make_async_copy(obufA.at[1 - q], out_hbm.at[:, ca_prev1, :],
                              oa_sems.at[1 - q]).wait()
        pltpu.make_async_copy(obufB.at[q], out_hbm.at[:, ca, :],
                              ob_sems.at[q]).wait()
        pltpu.make_async_copy(obufB.at[1 - q], out_hbm.at[:, ca1, :],
                              ob_sems.at[1 - q]).wait()


def kernel(x, t_emb, Ws1, bs1, Ws3, bs3, Ws2, bs2, W1, b1, W2, b2):
    B, T, C = x.shape
    N = B * T
    J = N // _R
    E, _, HR = W1.shape
    HS = Ws1.shape[1]
    f32 = jnp.float32

    x3 = x.reshape(J, _R, C)
    b1r = b1[:, None, :]
    b2r = b2[:, None, :]
    bs1r = bs1[None, :]
    bs3r = bs3[None, :]
    bs2r = bs2[None, :]

    out = pl.pallas_call(
        _body,
        grid=(_NG,),
        in_specs=[
            pl.BlockSpec(memory_space=pl.ANY),
            pl.BlockSpec((1, C, HR), lambda g: ((30 * g) % _R, 0, 0)),
            pl.BlockSpec((1, 1, HR), lambda g: ((30 * g) % _R, 0, 0)),
            pl.BlockSpec((1, HR, C), lambda g: ((30 * g) % _R, 0, 0)),
            pl.BlockSpec((1, 1, C), lambda g: ((30 * g) % _R, 0, 0)),
            pl.BlockSpec((1, C, HR), lambda g: ((30 * g + 47) % _R, 0, 0)),
            pl.BlockSpec((1, 1, HR), lambda g: ((30 * g + 47) % _R, 0, 0)),
            pl.BlockSpec((1, HR, C), lambda g: ((30 * g + 47) % _R, 0, 0)),
            pl.BlockSpec((1, 1, C), lambda g: ((30 * g + 47) % _R, 0, 0)),
            pl.BlockSpec((C, HS), lambda g: (0, 0)),
            pl.BlockSpec((1, HS), lambda g: (0, 0)),
            pl.BlockSpec((C, HS), lambda g: (0, 0)),
            pl.BlockSpec((1, HS), lambda g: (0, 0)),
            pl.BlockSpec((HS, C), lambda g: (0, 0)),
            pl.BlockSpec((1, C), lambda g: (0, 0)),
        ],
        out_specs=pl.BlockSpec(memory_space=pl.ANY),
        out_shape=jax.ShapeDtypeStruct((J, _R, C), f32),
        scratch_shapes=[
            pltpu.VMEM((5, J, C), f32),
            pltpu.VMEM((2, J, C), f32),
            pltpu.VMEM((2, J, C), f32),
            pltpu.VMEM((J, C), f32),
            pltpu.SemaphoreType.DMA((5,)),
            pltpu.SemaphoreType.DMA((2,)),
            pltpu.SemaphoreType.DMA((2,)),
        ],
    )(x3, W1, b1r, W2, b2r, W1, b1r, W2, b2r,
      Ws1, bs1r, Ws3, bs3r, Ws2, bs2r)

    return out.reshape(B, T, C)
